# baseline (speedup 1.0000x reference)
import functools

import jax
import jax.numpy as jnp
from jax import lax
from jax.experimental import pallas as pl
from jax.experimental.pallas import tpu as pltpu
from jax.experimental.pallas import tpu_sc as plsc

N_NODES = 100000
N_PAD = 102400
N_ROWS = N_PAD // 128
E = 1600000
E_PAD = 1605632
E_ROWS = E_PAD // 128
RPT = E_ROWS // 32
BR = 4
NB = RPT // BR
NPATCH = 1024
NPP = 1152
EM = 16384
EM_ROWS = EM // 128

NC, NS = 2, 16
_PROBE_NOSCALE = False
_SC_PARAMS = pltpu.CompilerParams(needs_layout_passes=False,
                                  use_tc_tiling_on_sc=False)


def _sc_mesh():
    return plsc.VectorSubcoreMesh(core_axis_name="c", subcore_axis_name="s",
                                  num_cores=NC, num_subcores=NS)


def _fill(ref, n_rows, width, value):
    @plsc.parallel_loop(0, n_rows, 1, unroll=8)
    def _(i):
        for k in range(width // 16):
            ref[i, pl.ds(16 * k, 16)] = jnp.full((16,), value, jnp.float32)


def _zero_acc(acc, s, rows_per_sub, zero_v, zrows, sem):
    base = s * rows_per_sub
    n_full = rows_per_sub // zrows

    def body(r, carry):
        pltpu.sync_copy(zero_v, acc.at[pl.ds(base + r * zrows, zrows)])
        return carry

    lax.fori_loop(0, n_full, body, 0)
    rem = rows_per_sub - n_full * zrows
    if rem:
        pltpu.sync_copy(zero_v.at[pl.ds(0, rem)],
                        acc.at[pl.ds(base + n_full * zrows, rem)])


def _dump_acc(acc, out, c, s, rows_per_sub):
    base = s * rows_per_sub
    pltpu.sync_copy(acc.at[pl.ds(base, rows_per_sub)],
                    out.at[c].at[pl.ds(base, rows_per_sub)])


def _sc_degrees_body(src2d, dst2d, gid2d,
                     deg_o, deg_i, cnt,
                     idx_v, ones_v, zero_v, acc, sem_a):
    c = lax.axis_index("c")
    s = lax.axis_index("s")
    w = c * NS + s
    _fill(ones_v, 128, 16, 1.0)
    _fill(zero_v, 128, 16, 0.0)

    def hist_pass(idx2d, out, tile_rows, nrows_acc):
        rps = nrows_acc // NS
        _zero_acc(acc, s, rps, zero_v, 128, sem_a)
        plsc.subcore_barrier()
        n_full = tile_rows // 8

        def batch(b, carry):
            pltpu.sync_copy(idx2d.at[pl.ds(w * tile_rows + b * 8, 8)], idx_v)
            descs = [pltpu.async_copy(ones_v, acc.at[idx_v.at[j]], sem_a,
                                      add=True) for j in range(8)]
            for d in descs:
                d.wait()
            return carry

        lax.fori_loop(0, n_full, batch, 0)
        rem = tile_rows - n_full * 8
        if rem:
            pltpu.sync_copy(
                idx2d.at[pl.ds(w * tile_rows + n_full * 8, rem)],
                idx_v.at[pl.ds(0, rem)])
            descs = [pltpu.async_copy(ones_v, acc.at[idx_v.at[j]], sem_a,
                                      add=True) for j in range(rem)]
            for d in descs:
                d.wait()
        plsc.subcore_barrier()
        _dump_acc(acc, out, c, s, rps)
        plsc.subcore_barrier()

    hist_pass(src2d, deg_o, RPT, N_PAD)
    hist_pass(dst2d, deg_i, RPT, N_PAD)
    hist_pass(gid2d, cnt, N_ROWS // 32, NPP)


def _make_sc_degrees():
    f32 = jnp.float32
    return pl.kernel(
        _sc_degrees_body,
        out_type=(
            jax.ShapeDtypeStruct((NC, N_PAD, 16), f32),
            jax.ShapeDtypeStruct((NC, N_PAD, 16), f32),
            jax.ShapeDtypeStruct((NC, NPP, 16), f32),
        ),
        mesh=_sc_mesh(),
        scratch_types=[
            pltpu.VMEM((8, 128), jnp.int32),
            pltpu.VMEM((128, 16), f32),
            pltpu.VMEM((128, 16), f32),
            pltpu.VMEM_SHARED((N_PAD, 16), f32),
            pltpu.SemaphoreType.DMA,
        ],
        compiler_params=_SC_PARAMS,
    )


def _make_sc_agg(nchunks):
    f32 = jnp.float32

    def body(*refs):
        tables = refs[:nchunks]
        srcf, dstf, ew = refs[nchunks:nchunks + 3]
        outs = refs[nchunks + 3:2 * nchunks + 3]
        (sidx_a, sidx_b, didx_a, didx_b, ew_a, ew_b, rows_a, rows_b,
         zero_v, acc, sem_g, sem_a) = refs[2 * nchunks + 3:]
        c = lax.axis_index("c")
        s = lax.axis_index("s")
        w = c * NS + s
        _fill(zero_v, 128, 16, 0.0)
        e0 = w * (RPT * 128)
        bsz = BR * 128

        def load(b, sidx, didx, ewv):
            pltpu.sync_copy(srcf.at[pl.ds(e0 + b * bsz, bsz)], sidx)
            pltpu.sync_copy(dstf.at[pl.ds(e0 + b * bsz, bsz)], didx)
            pltpu.sync_copy(ew.at[pl.ds(e0 + b * bsz, bsz)], ewv)

        def scale(rows, ewv):
            if _PROBE_NOSCALE:
                return

            @plsc.parallel_loop(0, BR * 128, 1, unroll=4)
            def _(e):
                sc = plsc.load_gather(ewv, [jnp.full((16,), e, jnp.int32)])
                rows[e, :] = rows[e, :] * sc

        for tbl, out in zip(tables, outs):
            def fire_g(sidx, rows):
                pltpu.async_copy(tbl.at[sidx], rows, sem_g)

            def drain_g(sidx, rows):
                pltpu.make_async_copy(tbl.at[sidx], rows, sem_g).wait()

            def fire_a(didx, rows):
                pltpu.async_copy(rows, acc.at[didx], sem_a, add=True)

            def drain_a(didx, rows):
                pltpu.make_async_copy(rows, acc.at[didx], sem_a).wait()

            _zero_acc(acc, s, N_PAD // NS, zero_v, 128, sem_a)
            plsc.subcore_barrier()

            load(0, sidx_a, didx_a, ew_a)
            fire_g(sidx_a, rows_a)

            def pair(bb, load_next):
                b0 = 2 * bb
                drain_g(sidx_a, rows_a)

                @pl.when(bb > 0)
                def _():
                    drain_a(didx_b, rows_b)

                load(b0 + 1, sidx_b, didx_b, ew_b)
                fire_g(sidx_b, rows_b)
                scale(rows_a, ew_a)
                fire_a(didx_a, rows_a)
                drain_g(sidx_b, rows_b)
                drain_a(didx_a, rows_a)
                if load_next:
                    load(b0 + 2, sidx_a, didx_a, ew_a)
                    fire_g(sidx_a, rows_a)
                scale(rows_b, ew_b)
                fire_a(didx_b, rows_b)

            def pair_body(bb, carry):
                pair(bb, True)
                return carry

            lax.fori_loop(0, NB // 2 - 1, pair_body, 0)
            pair(NB // 2 - 1, False)
            drain_a(didx_b, rows_b)
            plsc.subcore_barrier()
            _dump_acc(acc, out, c, s, N_PAD // NS)
            plsc.subcore_barrier()

    f32 = jnp.float32
    return pl.kernel(
        body,
        out_type=tuple(jax.ShapeDtypeStruct((NC, N_PAD, 16), f32)
                       for _ in range(nchunks)),
        mesh=_sc_mesh(),
        scratch_types=[
            pltpu.VMEM((BR * 128,), jnp.int32),
            pltpu.VMEM((BR * 128,), jnp.int32),
            pltpu.VMEM((BR * 128,), jnp.int32),
            pltpu.VMEM((BR * 128,), jnp.int32),
            pltpu.VMEM((BR * 128,), f32),
            pltpu.VMEM((BR * 128,), f32),
            pltpu.VMEM((BR * 128, 16), f32),
            pltpu.VMEM((BR * 128, 16), f32),
            pltpu.VMEM((128, 16), f32),
            pltpu.VMEM_SHARED((N_PAD, 16), f32),
            pltpu.SemaphoreType.DMA,
            pltpu.SemaphoreType.DMA,
        ],
        compiler_params=_SC_PARAMS,
    )


def _sc_readout_body(h2_3d, gid2d, out, idx_v, vals_v, zero_v, acc, sem_a):
    c = lax.axis_index("c")
    s = lax.axis_index("s")
    w = c * NS + s
    tile_rows = N_ROWS // 32

    @plsc.parallel_loop(0, 72, 1, unroll=8)
    def _(i):
        for k in range(4):
            zero_v[i, pl.ds(16 * k, 16)] = jnp.zeros((16,), jnp.float32)

    rps = NPP // NS
    base = s * rps
    pltpu.sync_copy(zero_v, acc.at[pl.ds(base, rps)])
    plsc.subcore_barrier()
    pltpu.sync_copy(gid2d.at[pl.ds(w * tile_rows, tile_rows)], idx_v)
    for b in range(5):
        pltpu.sync_copy(h2_3d.at[pl.ds(w * tile_rows + b * 5, 5)], vals_v)
        descs = [pltpu.async_copy(vals_v.at[j], acc.at[idx_v.at[b * 5 + j]],
                                  sem_a, add=True) for j in range(5)]
        for d in descs:
            d.wait()
    plsc.subcore_barrier()
    pltpu.sync_copy(acc.at[pl.ds(base, rps)], out.at[c].at[pl.ds(base, rps)])


def _make_sc_readout():
    f32 = jnp.float32
    return pl.kernel(
        _sc_readout_body,
        out_type=jax.ShapeDtypeStruct((NC, NPP, 64), f32),
        mesh=_sc_mesh(),
        scratch_types=[
            pltpu.VMEM((25, 128), jnp.int32),
            pltpu.VMEM((5, 128, 64), f32),
            pltpu.VMEM((72, 64), f32),
            pltpu.VMEM_SHARED((NPP, 64), f32),
            pltpu.SemaphoreType.DMA,
        ],
        compiler_params=_SC_PARAMS,
    )


BLK = 2048
GRID = N_PAD // BLK


def _leaky(x):
    return jnp.where(x >= 0, x, 0.01 * x)


def _norm(deg_ref):
    d = deg_ref[0, :, 0:1] + deg_ref[1, :, 0:1]
    return lax.rsqrt(jnp.maximum(d, 1.0))


def _k1_body(feats, dego, xn0, xn1):
    no = _norm(dego)
    xn = feats[...] * no
    xn0[...] = xn[:, :16]
    xn1[...] = jnp.concatenate(
        [xn[:, 16:18], jnp.zeros((BLK, 14), jnp.float32)], axis=1)


def _k2_body(s1p0, s1p1, dego, degi, w1, w2, t0, t1, t2, t3):
    ni = _norm(degi)
    no = _norm(dego)
    a = s1p0[0] + s1p0[1]
    b = s1p1[0] + s1p1[1]
    sfull = jnp.concatenate([a, b[:, :2]], axis=1)
    h = _leaky(jnp.dot(sfull * ni, w1[...],
                       preferred_element_type=jnp.float32))
    t = jnp.dot(h * no, w2[...], preferred_element_type=jnp.float32)
    t0[...] = t[:, 0:16]
    t1[...] = t[:, 16:32]
    t2[...] = t[:, 32:48]
    t3[...] = t[:, 48:64]


def _k3_body(p0, p1, p2, p3, degi, h2):
    ni = _norm(degi)
    agg = jnp.concatenate([p[0] + p[1] for p in (p0, p1, p2, p3)], axis=1)
    h2[...] = _leaky(agg * ni)


MEB = 512
MCH = EM // MEB


def _mesh_onehot(idx_row):
    cols = lax.broadcasted_iota(jnp.int32, (MEB, NPATCH), 1)
    return jnp.where(idx_row.reshape(MEB, 1) == cols, 1.0, 0.0)


def _tail_body(rsum, cnt, ms2d, md2d, wl, wc, w1m, w2m, wlm, blm, wcm,
               readouts, logits):
    f32 = jnp.float32
    sums = rsum[0, :NPATCH] + rsum[1, :NPATCH]
    c = cnt[0, :NPATCH, 0:1] + cnt[1, :NPATCH, 0:1]
    r = jnp.dot(jnp.dot(sums / jnp.maximum(c, 1.0), wl[...],
                        preferred_element_type=f32), wc[...],
                preferred_element_type=f32)
    readouts[...] = r

    mdo = jnp.zeros((NPATCH,), f32)
    mdi = jnp.zeros((NPATCH,), f32)
    for k in range(MCH):
        mdo = mdo + jnp.sum(_mesh_onehot(ms2d[k]), axis=0)
        mdi = mdi + jnp.sum(_mesh_onehot(md2d[k]), axis=0)
    mno = lax.rsqrt(jnp.maximum(mdo, 1.0)).reshape(NPATCH, 1)
    mni = lax.rsqrt(jnp.maximum(mdi, 1.0)).reshape(NPATCH, 1)

    def mesh_conv(x):
        agg = jnp.zeros((NPATCH, x.shape[1]), f32)
        for k in range(MCH):
            msgs = jnp.dot(_mesh_onehot(ms2d[k]), x, preferred_element_type=f32)
            agg = agg + jnp.dot(_mesh_onehot(md2d[k]).T, msgs,
                                preferred_element_type=f32)
        return agg

    xm = r * mno
    g1 = _leaky(mesh_conv(xm) * mni @ w1m[...])
    xm2 = (g1 * mno) @ w2m[...]
    g2 = _leaky(mesh_conv(xm2) * mni)
    z = jnp.dot(g2, wlm[...], preferred_element_type=f32) + blm[...]
    pooled = jnp.mean(z, axis=0, keepdims=True)
    logits[...] = jnp.dot(pooled, wcm[...], preferred_element_type=f32)


def _deg_spec():
    return pl.BlockSpec((2, BLK, 16), lambda i: (0, i, 0))


def _chunk_spec():
    return pl.BlockSpec((BLK, 16), lambda i: (i, 0))


def _pair_spec():
    return pl.BlockSpec((2, BLK, 16), lambda i: (0, i, 0))


def _make_k1():
    f32 = jnp.float32
    return pl.pallas_call(
        _k1_body,
        grid=(GRID,),
        in_specs=[pl.BlockSpec((BLK, 18), lambda i: (i, 0)), _deg_spec()],
        out_specs=[_chunk_spec(), _chunk_spec()],
        out_shape=[jax.ShapeDtypeStruct((N_PAD, 16), f32)] * 2,
    )


def _make_k2():
    f32 = jnp.float32
    return pl.pallas_call(
        _k2_body,
        grid=(GRID,),
        in_specs=[
            _pair_spec(), _pair_spec(), _deg_spec(), _deg_spec(),
            pl.BlockSpec((18, 128), lambda i: (0, 0)),
            pl.BlockSpec((128, 64), lambda i: (0, 0)),
        ],
        out_specs=[_chunk_spec()] * 4,
        out_shape=[jax.ShapeDtypeStruct((N_PAD, 16), f32)] * 4,
    )


def _make_k3():
    f32 = jnp.float32
    return pl.pallas_call(
        _k3_body,
        grid=(GRID,),
        in_specs=[_pair_spec()] * 4 + [_deg_spec()],
        out_specs=[pl.BlockSpec((BLK, 64), lambda i: (i, 0))],
        out_shape=[jax.ShapeDtypeStruct((N_PAD, 64), f32)],
    )


@jax.jit
def kernel(patch_feats, patch_edge_index, patch_edge_weight, patch_graph_ids,
           mesh_edge_index, W1p, W2p, Wlp, Wcp, W1m, W2m, Wlm, blm, Wcm):
    f32 = jnp.float32
    i32 = jnp.int32
    src = patch_edge_index[0].astype(i32)
    dst = patch_edge_index[1].astype(i32)
    pad_e = E_PAD - E
    dump = jnp.full((pad_e,), N_NODES, i32)
    src_f = jnp.concatenate([src, dump])
    dst_f = jnp.concatenate([dst, dump])
    src2d = src_f.reshape(E_ROWS, 128)
    dst2d = dst_f.reshape(E_ROWS, 128)
    ew_p = jnp.concatenate(
        [patch_edge_weight.astype(f32), jnp.zeros((pad_e,), f32)])
    gid2d = jnp.concatenate(
        [patch_graph_ids.astype(i32),
         jnp.full((N_PAD - N_NODES,), NPATCH, i32)]).reshape(N_ROWS, 128)
    ms2d = mesh_edge_index[0].astype(i32).reshape(MCH, MEB)
    md2d = mesh_edge_index[1].astype(i32).reshape(MCH, MEB)
    feats_pad = jnp.concatenate(
        [patch_feats, jnp.zeros((N_PAD - N_NODES, 18), f32)])

    deg_o, deg_i, cnt = _make_sc_degrees()(src2d, dst2d, gid2d)

    xn0, xn1 = _make_k1()(feats_pad, deg_o)
    s1p0, s1p1 = _make_sc_agg(2)(xn0, xn1, src_f, dst_f, ew_p)
    t0, t1, t2, t3 = _make_k2()(s1p0, s1p1, deg_o, deg_i, W1p, W2p)
    s2p = _make_sc_agg(4)(t0, t1, t2, t3, src_f, dst_f, ew_p)
    (h2,) = _make_k3()(*s2p, deg_i)

    rsum = _make_sc_readout()(h2.reshape(N_ROWS, 128, 64), gid2d)
    readouts, logits = pl.pallas_call(
        _tail_body,
        out_shape=[jax.ShapeDtypeStruct((NPATCH, 32), f32),
                   jax.ShapeDtypeStruct((1, 15), f32)],
        compiler_params=pltpu.CompilerParams(
            vmem_limit_bytes=100 * 1024 * 1024),
    )(rsum, cnt, ms2d, md2d, Wlp, Wcp, W1m, W2m, Wlm, blm.reshape(1, 16),
      Wcm)

    return (logits, readouts)


# speedup vs baseline: 1.2159x; 1.2159x over previous
import functools

import jax
import jax.numpy as jnp
from jax import lax
from jax.experimental import pallas as pl
from jax.experimental.pallas import tpu as pltpu
from jax.experimental.pallas import tpu_sc as plsc

N_NODES = 100000
N_PAD = 102400
N_ROWS = N_PAD // 128
E = 1600000
E_PAD = 1605632
E_ROWS = E_PAD // 128
RPT = E_ROWS // 32
BR = 4
NB = RPT // BR
NPATCH = 1024
NPP = 1152
EM = 16384
EM_ROWS = EM // 128

NC, NS = 2, 16
_PROBE_NOSCALE = False
_SC_PARAMS = pltpu.CompilerParams(needs_layout_passes=False,
                                  use_tc_tiling_on_sc=False)


def _sc_mesh():
    return plsc.VectorSubcoreMesh(core_axis_name="c", subcore_axis_name="s",
                                  num_cores=NC, num_subcores=NS)


def _fill(ref, n_rows, width, value):
    @plsc.parallel_loop(0, n_rows, 1, unroll=8)
    def _(i):
        for k in range(width // 16):
            ref[i, pl.ds(16 * k, 16)] = jnp.full((16,), value, jnp.float32)


def _zero_acc(acc, s, rows_per_sub, zero_v, zrows, sem):
    base = s * rows_per_sub
    n_full = rows_per_sub // zrows

    def body(r, carry):
        pltpu.sync_copy(zero_v, acc.at[pl.ds(base + r * zrows, zrows)])
        return carry

    lax.fori_loop(0, n_full, body, 0)
    rem = rows_per_sub - n_full * zrows
    if rem:
        pltpu.sync_copy(zero_v.at[pl.ds(0, rem)],
                        acc.at[pl.ds(base + n_full * zrows, rem)])


def _dump_acc(acc, out, c, s, rows_per_sub):
    base = s * rows_per_sub
    pltpu.sync_copy(acc.at[pl.ds(base, rows_per_sub)],
                    out.at[c].at[pl.ds(base, rows_per_sub)])


def _sc_degrees_body(src2d, dst2d, gid2d,
                     deg_o, deg_i, cnt,
                     idx_v, ones_v, zero_v, acc, sem_a):
    c = lax.axis_index("c")
    s = lax.axis_index("s")
    w = c * NS + s
    _fill(ones_v, 128, 16, 1.0)
    _fill(zero_v, 128, 16, 0.0)

    def hist_pass(idx2d, out, tile_rows, nrows_acc):
        rps = nrows_acc // NS
        _zero_acc(acc, s, rps, zero_v, 128, sem_a)
        plsc.subcore_barrier()
        n_full = tile_rows // 8

        def batch(b, carry):
            pltpu.sync_copy(idx2d.at[pl.ds(w * tile_rows + b * 8, 8)], idx_v)
            descs = [pltpu.async_copy(ones_v, acc.at[idx_v.at[j]], sem_a,
                                      add=True) for j in range(8)]
            for d in descs:
                d.wait()
            return carry

        lax.fori_loop(0, n_full, batch, 0)
        rem = tile_rows - n_full * 8
        if rem:
            pltpu.sync_copy(
                idx2d.at[pl.ds(w * tile_rows + n_full * 8, rem)],
                idx_v.at[pl.ds(0, rem)])
            descs = [pltpu.async_copy(ones_v, acc.at[idx_v.at[j]], sem_a,
                                      add=True) for j in range(rem)]
            for d in descs:
                d.wait()
        plsc.subcore_barrier()
        _dump_acc(acc, out, c, s, rps)
        plsc.subcore_barrier()

    hist_pass(src2d, deg_o, RPT, N_PAD)
    hist_pass(dst2d, deg_i, RPT, N_PAD)
    hist_pass(gid2d, cnt, N_ROWS // 32, NPP)


def _make_sc_degrees():
    f32 = jnp.float32
    return pl.kernel(
        _sc_degrees_body,
        out_type=(
            jax.ShapeDtypeStruct((NC, N_PAD, 16), f32),
            jax.ShapeDtypeStruct((NC, N_PAD, 16), f32),
            jax.ShapeDtypeStruct((NC, NPP, 16), f32),
        ),
        mesh=_sc_mesh(),
        scratch_types=[
            pltpu.VMEM((8, 128), jnp.int32),
            pltpu.VMEM((128, 16), f32),
            pltpu.VMEM((128, 16), f32),
            pltpu.VMEM_SHARED((N_PAD, 16), f32),
            pltpu.SemaphoreType.DMA,
        ],
        compiler_params=_SC_PARAMS,
    )


def _make_sc_agg(nchunks):
    f32 = jnp.float32

    def body(*refs):
        tables = refs[:nchunks]
        srcf, dstf, ew = refs[nchunks:nchunks + 3]
        outs = refs[nchunks + 3:2 * nchunks + 3]
        (sidx_a, sidx_b, didx_a, didx_b, ew_a, ew_b, rows_a, rows_b,
         zero_v, acc, sem_g, sem_a) = refs[2 * nchunks + 3:]
        c = lax.axis_index("c")
        s = lax.axis_index("s")
        w = c * NS + s
        _fill(zero_v, 128, 16, 0.0)
        e0 = w * (RPT * 128)
        bsz = BR * 128

        def load(b, sidx, didx, ewv):
            pltpu.sync_copy(srcf.at[pl.ds(e0 + b * bsz, bsz)], sidx)
            pltpu.sync_copy(dstf.at[pl.ds(e0 + b * bsz, bsz)], didx)
            pltpu.sync_copy(ew.at[pl.ds(e0 + b * bsz, bsz)], ewv)

        def scale(rows, ewv):
            if _PROBE_NOSCALE:
                return

            @plsc.parallel_loop(0, BR * 128, 1, unroll=4)
            def _(e):
                sc = plsc.load_gather(ewv, [jnp.full((16,), e, jnp.int32)])
                rows[e, :] = rows[e, :] * sc

        for tbl, out in zip(tables, outs):
            def fire_g(sidx, rows):
                pltpu.async_copy(tbl.at[sidx], rows, sem_g)

            def drain_g(sidx, rows):
                pltpu.make_async_copy(tbl.at[sidx], rows, sem_g).wait()

            def fire_a(didx, rows):
                pltpu.async_copy(rows, acc.at[didx], sem_a, add=True)

            def drain_a(didx, rows):
                pltpu.make_async_copy(rows, acc.at[didx], sem_a).wait()

            _zero_acc(acc, s, N_PAD // NS, zero_v, 128, sem_a)
            plsc.subcore_barrier()

            load(0, sidx_a, didx_a, ew_a)
            fire_g(sidx_a, rows_a)

            def pair(bb, load_next):
                b0 = 2 * bb
                drain_g(sidx_a, rows_a)

                @pl.when(bb > 0)
                def _():
                    drain_a(didx_b, rows_b)

                load(b0 + 1, sidx_b, didx_b, ew_b)
                fire_g(sidx_b, rows_b)
                scale(rows_a, ew_a)
                fire_a(didx_a, rows_a)
                drain_g(sidx_b, rows_b)
                drain_a(didx_a, rows_a)
                if load_next:
                    load(b0 + 2, sidx_a, didx_a, ew_a)
                    fire_g(sidx_a, rows_a)
                scale(rows_b, ew_b)
                fire_a(didx_b, rows_b)

            def pair_body(bb, carry):
                pair(bb, True)
                return carry

            lax.fori_loop(0, NB // 2 - 1, pair_body, 0)
            pair(NB // 2 - 1, False)
            drain_a(didx_b, rows_b)
            plsc.subcore_barrier()
            _dump_acc(acc, out, c, s, N_PAD // NS)
            plsc.subcore_barrier()

    f32 = jnp.float32
    return pl.kernel(
        body,
        out_type=tuple(jax.ShapeDtypeStruct((NC, N_PAD, 16), f32)
                       for _ in range(nchunks)),
        mesh=_sc_mesh(),
        scratch_types=[
            pltpu.VMEM((BR * 128,), jnp.int32),
            pltpu.VMEM((BR * 128,), jnp.int32),
            pltpu.VMEM((BR * 128,), jnp.int32),
            pltpu.VMEM((BR * 128,), jnp.int32),
            pltpu.VMEM((BR * 128,), f32),
            pltpu.VMEM((BR * 128,), f32),
            pltpu.VMEM((BR * 128, 16), f32),
            pltpu.VMEM((BR * 128, 16), f32),
            pltpu.VMEM((128, 16), f32),
            pltpu.VMEM_SHARED((N_PAD, 16), f32),
            pltpu.SemaphoreType.DMA,
            pltpu.SemaphoreType.DMA,
        ],
        compiler_params=_SC_PARAMS,
    )


def _sc_readout_body(h0, h1, h2, h3, gid2d, out,
                     idx_v, vals_v, zero_v, a0, a1, a2, a3, sem_a):
    c = lax.axis_index("c")
    s = lax.axis_index("s")
    w = c * NS + s
    tile_rows = N_ROWS // 32
    _fill(zero_v, 72, 16, 0.0)
    rps = NPP // NS
    base = s * rps
    accs = (a0, a1, a2, a3)
    for acc in accs:
        pltpu.sync_copy(zero_v, acc.at[pl.ds(base, rps)])
    plsc.subcore_barrier()
    pltpu.sync_copy(gid2d.at[pl.ds(w * tile_rows, tile_rows)], idx_v)
    for hsrc, acc in zip((h0, h1, h2, h3), accs):
        for b in range(5):
            pltpu.sync_copy(hsrc.at[pl.ds(w * tile_rows + b * 5, 5)], vals_v)
            descs = [pltpu.async_copy(vals_v.at[j],
                                      acc.at[idx_v.at[b * 5 + j]],
                                      sem_a, add=True) for j in range(5)]
            for d in descs:
                d.wait()
    plsc.subcore_barrier()
    for ch, acc in enumerate(accs):
        pltpu.sync_copy(acc.at[pl.ds(base, rps)],
                        out.at[c].at[ch].at[pl.ds(base, rps)])


def _make_sc_readout():
    f32 = jnp.float32
    return pl.kernel(
        _sc_readout_body,
        out_type=jax.ShapeDtypeStruct((NC, 4, NPP, 16), f32),
        mesh=_sc_mesh(),
        scratch_types=[
            pltpu.VMEM((25, 128), jnp.int32),
            pltpu.VMEM((5, 128, 16), f32),
            pltpu.VMEM((72, 16), f32),
            pltpu.VMEM_SHARED((NPP, 16), f32),
            pltpu.VMEM_SHARED((NPP, 16), f32),
            pltpu.VMEM_SHARED((NPP, 16), f32),
            pltpu.VMEM_SHARED((NPP, 16), f32),
            pltpu.SemaphoreType.DMA,
        ],
        compiler_params=_SC_PARAMS,
    )


BLK = 2048
GRID = N_PAD // BLK


def _leaky(x):
    return jnp.where(x >= 0, x, 0.01 * x)


def _norm(deg_ref):
    d = deg_ref[0, :, 0:1] + deg_ref[1, :, 0:1]
    return lax.rsqrt(jnp.maximum(d, 1.0))


def _k1_body(feats, dego, xn0, xn1):
    no = _norm(dego)
    xn = feats[...] * no
    xn0[...] = xn[:, :16]
    xn1[...] = jnp.concatenate(
        [xn[:, 16:18], jnp.zeros((BLK, 14), jnp.float32)], axis=1)


def _norm_view(deg_ref):
    return lax.rsqrt(jnp.maximum(deg_ref[0] + deg_ref[1], 1.0))


def _k2_body(s1p0v, s1p1v, degiv, dego, k1a, k1b, w2, t0, t1, t2, t3):
    f32 = jnp.float32
    ni = _norm_view(degiv)
    h_v = jnp.dot((s1p0v[0] + s1p0v[1]) * ni, k1a[...],
                  preferred_element_type=f32)
    h_v = h_v + jnp.dot((s1p1v[0] + s1p1v[1]) * ni, k1b[...],
                        preferred_element_type=f32)
    h = _leaky(h_v.reshape(BLK, 128))
    no = _norm(dego)
    t = jnp.dot(h * no, w2[...], preferred_element_type=f32)
    t0[...] = t[:, 0:16]
    t1[...] = t[:, 16:32]
    t2[...] = t[:, 32:48]
    t3[...] = t[:, 48:64]


def _k3_body(p0, p1, p2, p3, degiv, h0, h1, h2, h3):
    ni = _norm_view(degiv)
    for p, o in ((p0, h0), (p1, h1), (p2, h2), (p3, h3)):
        o[...] = _leaky((p[0] + p[1]) * ni)


MEB = 512
MCH = EM // MEB


def _mesh_onehot(idx_row):
    cols = lax.broadcasted_iota(jnp.int32, (MEB, NPATCH), 1)
    return jnp.where(idx_row.reshape(MEB, 1) == cols, 1.0, 0.0)


def _tail_body(rsum, cnt, ms2d, md2d, wl, wc, w1m, w2m, wlm, blm, wcm,
               readouts, logits):
    f32 = jnp.float32
    sums = jnp.concatenate(
        [rsum[0, ch, :NPATCH] + rsum[1, ch, :NPATCH] for ch in range(4)],
        axis=1)
    c = cnt[0, :NPATCH, 0:1] + cnt[1, :NPATCH, 0:1]
    r = jnp.dot(jnp.dot(sums / jnp.maximum(c, 1.0), wl[...],
                        preferred_element_type=f32), wc[...],
                preferred_element_type=f32)
    readouts[...] = r

    mdo = jnp.zeros((NPATCH,), f32)
    mdi = jnp.zeros((NPATCH,), f32)
    for k in range(MCH):
        mdo = mdo + jnp.sum(_mesh_onehot(ms2d[k]), axis=0)
        mdi = mdi + jnp.sum(_mesh_onehot(md2d[k]), axis=0)
    mno = lax.rsqrt(jnp.maximum(mdo, 1.0)).reshape(NPATCH, 1)
    mni = lax.rsqrt(jnp.maximum(mdi, 1.0)).reshape(NPATCH, 1)

    def mesh_conv(x):
        agg = jnp.zeros((NPATCH, x.shape[1]), f32)
        for k in range(MCH):
            msgs = jnp.dot(_mesh_onehot(ms2d[k]), x, preferred_element_type=f32)
            agg = agg + jnp.dot(_mesh_onehot(md2d[k]).T, msgs,
                                preferred_element_type=f32)
        return agg

    xm = r * mno
    g1 = _leaky(mesh_conv(xm) * mni @ w1m[...])
    xm2 = (g1 * mno) @ w2m[...]
    g2 = _leaky(mesh_conv(xm2) * mni)
    z = jnp.dot(g2, wlm[...], preferred_element_type=f32) + blm[...]
    pooled = jnp.mean(z, axis=0, keepdims=True)
    logits[...] = jnp.dot(pooled, wcm[...], preferred_element_type=f32)


def _deg_spec():
    return pl.BlockSpec((2, BLK, 16), lambda i: (0, i, 0))


def _chunk_spec():
    return pl.BlockSpec((BLK, 16), lambda i: (i, 0))


def _pair_spec():
    return pl.BlockSpec((2, BLK, 16), lambda i: (0, i, 0))


def _make_k1():
    f32 = jnp.float32
    return pl.pallas_call(
        _k1_body,
        grid=(GRID,),
        in_specs=[pl.BlockSpec((BLK, 18), lambda i: (i, 0)), _deg_spec()],
        out_specs=[_chunk_spec(), _chunk_spec()],
        out_shape=[jax.ShapeDtypeStruct((N_PAD, 16), f32)] * 2,
    )


VBLK = BLK // 8
NV = N_PAD // 8


def _view_spec():
    return pl.BlockSpec((2, VBLK, 128), lambda i: (0, i, 0))


def _vout_spec():
    return pl.BlockSpec((VBLK, 128), lambda i: (i, 0))


def _make_k2():
    f32 = jnp.float32
    return pl.pallas_call(
        _k2_body,
        grid=(GRID,),
        in_specs=[
            _view_spec(), _view_spec(), _view_spec(), _deg_spec(),
            pl.BlockSpec((128, 1024), lambda i: (0, 0)),
            pl.BlockSpec((128, 1024), lambda i: (0, 0)),
            pl.BlockSpec((128, 64), lambda i: (0, 0)),
        ],
        out_specs=[_chunk_spec()] * 4,
        out_shape=[jax.ShapeDtypeStruct((N_PAD, 16), f32)] * 4,
    )


def _make_k3():
    f32 = jnp.float32
    return pl.pallas_call(
        _k3_body,
        grid=(GRID,),
        in_specs=[_view_spec()] * 5,
        out_specs=[_vout_spec()] * 4,
        out_shape=[jax.ShapeDtypeStruct((NV, 128), f32)] * 4,
    )


@jax.jit
def kernel(patch_feats, patch_edge_index, patch_edge_weight, patch_graph_ids,
           mesh_edge_index, W1p, W2p, Wlp, Wcp, W1m, W2m, Wlm, blm, Wcm):
    f32 = jnp.float32
    i32 = jnp.int32
    src = patch_edge_index[0].astype(i32)
    dst = patch_edge_index[1].astype(i32)
    pad_e = E_PAD - E
    dump = jnp.full((pad_e,), N_NODES, i32)
    src_f = jnp.concatenate([src, dump])
    dst_f = jnp.concatenate([dst, dump])
    src2d = src_f.reshape(E_ROWS, 128)
    dst2d = dst_f.reshape(E_ROWS, 128)
    ew_p = jnp.concatenate(
        [patch_edge_weight.astype(f32), jnp.zeros((pad_e,), f32)])
    gid2d = jnp.concatenate(
        [patch_graph_ids.astype(i32),
         jnp.full((N_PAD - N_NODES,), NPATCH, i32)]).reshape(N_ROWS, 128)
    ms2d = mesh_edge_index[0].astype(i32).reshape(MCH, MEB)
    md2d = mesh_edge_index[1].astype(i32).reshape(MCH, MEB)
    feats_pad = jnp.concatenate(
        [patch_feats, jnp.zeros((N_PAD - N_NODES, 18), f32)])

    deg_o, deg_i, cnt = _make_sc_degrees()(src2d, dst2d, gid2d)
    degi_v = deg_i.reshape(2, NV, 128)

    xn0, xn1 = _make_k1()(feats_pad, deg_o)
    s1p0, s1p1 = _make_sc_agg(2)(xn0, xn1, src_f, dst_f, ew_p)
    eye8 = jnp.eye(8, dtype=f32)
    k1a = jnp.kron(eye8, W1p[:16])
    k1b = jnp.kron(eye8, jnp.concatenate(
        [W1p[16:18], jnp.zeros((14, 128), f32)]))
    t0, t1, t2, t3 = _make_k2()(
        s1p0.reshape(2, NV, 128), s1p1.reshape(2, NV, 128),
        degi_v, deg_o, k1a, k1b, W2p)
    s2p = _make_sc_agg(4)(t0, t1, t2, t3, src_f, dst_f, ew_p)
    h2v = _make_k3()(*[p.reshape(2, NV, 128) for p in s2p], degi_v)

    rsum = _make_sc_readout()(
        *[h.reshape(N_ROWS, 128, 16) for h in h2v], gid2d)
    readouts, logits = pl.pallas_call(
        _tail_body,
        out_shape=[jax.ShapeDtypeStruct((NPATCH, 32), f32),
                   jax.ShapeDtypeStruct((1, 15), f32)],
        compiler_params=pltpu.CompilerParams(
            vmem_limit_bytes=100 * 1024 * 1024),
    )(rsum, cnt, ms2d, md2d, Wlp, Wcp, W1m, W2m, Wlm, blm.reshape(1, 16),
      Wcm)

    return (logits, readouts)


# speedup vs baseline: 1.2995x; 1.0688x over previous
import functools

import jax
import jax.numpy as jnp
from jax import lax
from jax.experimental import pallas as pl
from jax.experimental.pallas import tpu as pltpu
from jax.experimental.pallas import tpu_sc as plsc

N_NODES = 100000
N_PAD = 102400
N_ROWS = N_PAD // 128
E = 1600000
E_PAD = 1605632
E_ROWS = E_PAD // 128
RPT = E_ROWS // 32
BR = 4
NB = RPT // BR
NPATCH = 1024
NPP = 1152
EM = 16384
EM_ROWS = EM // 128

NC, NS = 2, 16
_PROBE_NOSCALE = False
_SC_PARAMS = pltpu.CompilerParams(needs_layout_passes=False,
                                  use_tc_tiling_on_sc=False)


def _sc_mesh():
    return plsc.VectorSubcoreMesh(core_axis_name="c", subcore_axis_name="s",
                                  num_cores=NC, num_subcores=NS)


def _fill(ref, n_rows, width, value):
    @plsc.parallel_loop(0, n_rows, 1, unroll=8)
    def _(i):
        for k in range(width // 16):
            ref[i, pl.ds(16 * k, 16)] = jnp.full((16,), value, jnp.float32)


def _zero_acc(acc, s, rows_per_sub, zero_v, zrows, sem):
    base = s * rows_per_sub
    n_full = rows_per_sub // zrows

    def body(r, carry):
        pltpu.sync_copy(zero_v, acc.at[pl.ds(base + r * zrows, zrows)])
        return carry

    lax.fori_loop(0, n_full, body, 0)
    rem = rows_per_sub - n_full * zrows
    if rem:
        pltpu.sync_copy(zero_v.at[pl.ds(0, rem)],
                        acc.at[pl.ds(base + n_full * zrows, rem)])


def _dump_acc(acc, out, c, s, rows_per_sub):
    base = s * rows_per_sub
    pltpu.sync_copy(acc.at[pl.ds(base, rows_per_sub)],
                    out.at[c].at[pl.ds(base, rows_per_sub)])


def _sc_degrees_body(src2d, dst2d, gid2d,
                     deg_o, deg_i, cnt,
                     idx_v, ones_v, zero_v, acc, sem_a):
    c = lax.axis_index("c")
    s = lax.axis_index("s")
    w = c * NS + s
    _fill(ones_v, 128, 16, 1.0)
    _fill(zero_v, 128, 16, 0.0)

    def hist_pass(idx2d, out, tile_rows, nrows_acc):
        rps = nrows_acc // NS
        _zero_acc(acc, s, rps, zero_v, 128, sem_a)
        plsc.subcore_barrier()
        n_full = tile_rows // 8

        def batch(b, carry):
            pltpu.sync_copy(idx2d.at[pl.ds(w * tile_rows + b * 8, 8)], idx_v)
            descs = [pltpu.async_copy(ones_v, acc.at[idx_v.at[j]], sem_a,
                                      add=True) for j in range(8)]
            for d in descs:
                d.wait()
            return carry

        lax.fori_loop(0, n_full, batch, 0)
        rem = tile_rows - n_full * 8
        if rem:
            pltpu.sync_copy(
                idx2d.at[pl.ds(w * tile_rows + n_full * 8, rem)],
                idx_v.at[pl.ds(0, rem)])
            descs = [pltpu.async_copy(ones_v, acc.at[idx_v.at[j]], sem_a,
                                      add=True) for j in range(rem)]
            for d in descs:
                d.wait()
        plsc.subcore_barrier()
        _dump_acc(acc, out, c, s, rps)
        plsc.subcore_barrier()

    hist_pass(src2d, deg_o, RPT, N_PAD)
    hist_pass(dst2d, deg_i, RPT, N_PAD)
    hist_pass(gid2d, cnt, N_ROWS // 32, NPP)


def _make_sc_degrees():
    f32 = jnp.float32
    return pl.kernel(
        _sc_degrees_body,
        out_type=(
            jax.ShapeDtypeStruct((NC, N_PAD, 16), f32),
            jax.ShapeDtypeStruct((NC, N_PAD, 16), f32),
            jax.ShapeDtypeStruct((NC, NPP, 16), f32),
        ),
        mesh=_sc_mesh(),
        scratch_types=[
            pltpu.VMEM((8, 128), jnp.int32),
            pltpu.VMEM((128, 16), f32),
            pltpu.VMEM((128, 16), f32),
            pltpu.VMEM_SHARED((N_PAD, 16), f32),
            pltpu.SemaphoreType.DMA,
        ],
        compiler_params=_SC_PARAMS,
    )


def _make_sc_agg(nchunks):
    f32 = jnp.float32

    def body(*refs):
        tables = refs[:nchunks]
        srcf, dstf, ew = refs[nchunks:nchunks + 3]
        outs = refs[nchunks + 3:2 * nchunks + 3]
        (sidx_a, sidx_b, didx_a, didx_b, ew_a, ew_b, rows_a, rows_b,
         zero_v, acc, sem_g, sem_a) = refs[2 * nchunks + 3:]
        c = lax.axis_index("c")
        s = lax.axis_index("s")
        w = c * NS + s
        _fill(zero_v, 128, 16, 0.0)
        e0 = w * (RPT * 128)
        bsz = BR * 128

        def load(b, sidx, didx, ewv):
            pltpu.sync_copy(srcf.at[pl.ds(e0 + b * bsz, bsz)], sidx)
            pltpu.sync_copy(dstf.at[pl.ds(e0 + b * bsz, bsz)], didx)
            pltpu.sync_copy(ew.at[pl.ds(e0 + b * bsz, bsz)], ewv)

        def scale(rows, ewv):
            if _PROBE_NOSCALE:
                return

            @plsc.parallel_loop(0, BR * 128, 1, unroll=4)
            def _(e):
                sc = plsc.load_gather(ewv, [jnp.full((16,), e, jnp.int32)])
                rows[e, :] = rows[e, :] * sc

        for tbl, out in zip(tables, outs):
            def fire_g(sidx, rows):
                pltpu.async_copy(tbl.at[sidx], rows, sem_g)

            def drain_g(sidx, rows):
                pltpu.make_async_copy(tbl.at[sidx], rows, sem_g).wait()

            def fire_a(didx, rows):
                pltpu.async_copy(rows, acc.at[didx], sem_a, add=True)

            def drain_a(didx, rows):
                pltpu.make_async_copy(rows, acc.at[didx], sem_a).wait()

            _zero_acc(acc, s, N_PAD // NS, zero_v, 128, sem_a)
            plsc.subcore_barrier()

            load(0, sidx_a, didx_a, ew_a)
            fire_g(sidx_a, rows_a)

            def pair(bb, load_next):
                b0 = 2 * bb
                drain_g(sidx_a, rows_a)

                @pl.when(bb > 0)
                def _():
                    drain_a(didx_b, rows_b)

                load(b0 + 1, sidx_b, didx_b, ew_b)
                fire_g(sidx_b, rows_b)
                scale(rows_a, ew_a)
                fire_a(didx_a, rows_a)
                drain_g(sidx_b, rows_b)
                drain_a(didx_a, rows_a)
                if load_next:
                    load(b0 + 2, sidx_a, didx_a, ew_a)
                    fire_g(sidx_a, rows_a)
                scale(rows_b, ew_b)
                fire_a(didx_b, rows_b)

            def pair_body(bb, carry):
                pair(bb, True)
                return carry

            lax.fori_loop(0, NB // 2 - 1, pair_body, 0)
            pair(NB // 2 - 1, False)
            drain_a(didx_b, rows_b)
            plsc.subcore_barrier()
            _dump_acc(acc, out, c, s, N_PAD // NS)
            plsc.subcore_barrier()

    f32 = jnp.float32
    return pl.kernel(
        body,
        out_type=tuple(jax.ShapeDtypeStruct((NC, N_PAD, 16), f32)
                       for _ in range(nchunks)),
        mesh=_sc_mesh(),
        scratch_types=[
            pltpu.VMEM((BR * 128,), jnp.int32),
            pltpu.VMEM((BR * 128,), jnp.int32),
            pltpu.VMEM((BR * 128,), jnp.int32),
            pltpu.VMEM((BR * 128,), jnp.int32),
            pltpu.VMEM((BR * 128,), f32),
            pltpu.VMEM((BR * 128,), f32),
            pltpu.VMEM((BR * 128, 16), f32),
            pltpu.VMEM((BR * 128, 16), f32),
            pltpu.VMEM((128, 16), f32),
            pltpu.VMEM_SHARED((N_PAD, 16), f32),
            pltpu.SemaphoreType.DMA,
            pltpu.SemaphoreType.DMA,
        ],
        compiler_params=_SC_PARAMS,
    )


def _sc_readout_body(h0, h1, h2, h3, gid2d, out,
                     idx_v, vals_v, zero_v, a0, a1, a2, a3, sem_a):
    c = lax.axis_index("c")
    s = lax.axis_index("s")
    w = c * NS + s
    tile_rows = N_ROWS // 32
    _fill(zero_v, 72, 16, 0.0)
    rps = NPP // NS
    base = s * rps
    accs = (a0, a1, a2, a3)
    for acc in accs:
        pltpu.sync_copy(zero_v, acc.at[pl.ds(base, rps)])
    plsc.subcore_barrier()
    pltpu.sync_copy(gid2d.at[pl.ds(w * tile_rows, tile_rows)], idx_v)
    for hsrc, acc in zip((h0, h1, h2, h3), accs):
        for b in range(5):
            pltpu.sync_copy(hsrc.at[pl.ds(w * tile_rows + b * 5, 5)], vals_v)
            descs = [pltpu.async_copy(vals_v.at[j],
                                      acc.at[idx_v.at[b * 5 + j]],
                                      sem_a, add=True) for j in range(5)]
            for d in descs:
                d.wait()
    plsc.subcore_barrier()
    for ch, acc in enumerate(accs):
        pltpu.sync_copy(acc.at[pl.ds(base, rps)],
                        out.at[c].at[ch].at[pl.ds(base, rps)])


def _make_sc_readout():
    f32 = jnp.float32
    return pl.kernel(
        _sc_readout_body,
        out_type=jax.ShapeDtypeStruct((NC, 4, NPP, 16), f32),
        mesh=_sc_mesh(),
        scratch_types=[
            pltpu.VMEM((25, 128), jnp.int32),
            pltpu.VMEM((5, 128, 16), f32),
            pltpu.VMEM((72, 16), f32),
            pltpu.VMEM_SHARED((NPP, 16), f32),
            pltpu.VMEM_SHARED((NPP, 16), f32),
            pltpu.VMEM_SHARED((NPP, 16), f32),
            pltpu.VMEM_SHARED((NPP, 16), f32),
            pltpu.SemaphoreType.DMA,
        ],
        compiler_params=_SC_PARAMS,
    )


BLK = 2048
GRID = N_PAD // BLK


def _leaky(x):
    return jnp.where(x >= 0, x, 0.01 * x)


def _norm(deg_ref):
    d = deg_ref[0, :, 0:1] + deg_ref[1, :, 0:1]
    return lax.rsqrt(jnp.maximum(d, 1.0))


def _k1_body(feats, dego, xn0, xn1):
    no = _norm(dego)
    xn = feats[...] * no
    xn0[...] = xn[:, :16]
    xn1[...] = jnp.concatenate(
        [xn[:, 16:18], jnp.zeros((BLK, 14), jnp.float32)], axis=1)


def _norm_view(deg_ref):
    return lax.rsqrt(jnp.maximum(deg_ref[0] + deg_ref[1], 1.0))


def _k2_body(s1p0v, s1p1v, degiv, degov, k1a, k1b, rsel, kw2, t0, t1, t2, t3):
    f32 = jnp.float32
    ni = _norm_view(degiv)
    h_v = jnp.dot((s1p0v[0] + s1p0v[1]) * ni, k1a[...],
                  preferred_element_type=f32)
    h_v = h_v + jnp.dot((s1p1v[0] + s1p1v[1]) * ni, k1b[...],
                        preferred_element_type=f32)
    no_v = jnp.dot(_norm_view(degov), rsel[...], preferred_element_type=f32)
    t_v = jnp.dot(_leaky(h_v) * no_v, kw2[...], preferred_element_type=f32)
    t0[...] = t_v[:, 0:128]
    t1[...] = t_v[:, 128:256]
    t2[...] = t_v[:, 256:384]
    t3[...] = t_v[:, 384:512]


def _k3_body(p0, p1, p2, p3, degiv, h0, h1, h2, h3):
    ni = _norm_view(degiv)
    for p, o in ((p0, h0), (p1, h1), (p2, h2), (p3, h3)):
        o[...] = _leaky((p[0] + p[1]) * ni)


MEB = 512
MCH = EM // MEB


def _mesh_onehot(idx_row):
    cols = lax.broadcasted_iota(jnp.int32, (MEB, NPATCH), 1)
    return jnp.where(idx_row.reshape(MEB, 1) == cols, 1.0, 0.0)


def _tail_body(rsum, cnt, ms2d, md2d, wl, wc, w1m, w2m, wlm, blm, wcm,
               readouts, logits):
    f32 = jnp.float32
    sums = jnp.concatenate(
        [rsum[0, ch, :NPATCH] + rsum[1, ch, :NPATCH] for ch in range(4)],
        axis=1)
    c = cnt[0, :NPATCH, 0:1] + cnt[1, :NPATCH, 0:1]
    r = jnp.dot(jnp.dot(sums / jnp.maximum(c, 1.0), wl[...],
                        preferred_element_type=f32), wc[...],
                preferred_element_type=f32)
    readouts[...] = r

    mdo = jnp.zeros((NPATCH,), f32)
    mdi = jnp.zeros((NPATCH,), f32)
    for k in range(MCH):
        mdo = mdo + jnp.sum(_mesh_onehot(ms2d[k]), axis=0)
        mdi = mdi + jnp.sum(_mesh_onehot(md2d[k]), axis=0)
    mno = lax.rsqrt(jnp.maximum(mdo, 1.0)).reshape(NPATCH, 1)
    mni = lax.rsqrt(jnp.maximum(mdi, 1.0)).reshape(NPATCH, 1)

    def mesh_conv(x):
        agg = jnp.zeros((NPATCH, x.shape[1]), f32)
        for k in range(MCH):
            msgs = jnp.dot(_mesh_onehot(ms2d[k]), x, preferred_element_type=f32)
            agg = agg + jnp.dot(_mesh_onehot(md2d[k]).T, msgs,
                                preferred_element_type=f32)
        return agg

    xm = r * mno
    g1 = _leaky(mesh_conv(xm) * mni @ w1m[...])
    xm2 = (g1 * mno) @ w2m[...]
    g2 = _leaky(mesh_conv(xm2) * mni)
    z = jnp.dot(g2, wlm[...], preferred_element_type=f32) + blm[...]
    pooled = jnp.mean(z, axis=0, keepdims=True)
    logits[...] = jnp.dot(pooled, wcm[...], preferred_element_type=f32)


def _deg_spec():
    return pl.BlockSpec((2, BLK, 16), lambda i: (0, i, 0))


def _chunk_spec():
    return pl.BlockSpec((BLK, 16), lambda i: (i, 0))


def _pair_spec():
    return pl.BlockSpec((2, BLK, 16), lambda i: (0, i, 0))


def _make_k1():
    f32 = jnp.float32
    return pl.pallas_call(
        _k1_body,
        grid=(GRID,),
        in_specs=[pl.BlockSpec((BLK, 18), lambda i: (i, 0)), _deg_spec()],
        out_specs=[_chunk_spec(), _chunk_spec()],
        out_shape=[jax.ShapeDtypeStruct((N_PAD, 16), f32)] * 2,
    )


VBLK = BLK // 8
NV = N_PAD // 8


def _view_spec():
    return pl.BlockSpec((2, VBLK, 128), lambda i: (0, i, 0))


def _vout_spec():
    return pl.BlockSpec((VBLK, 128), lambda i: (i, 0))


def _make_k2():
    f32 = jnp.float32
    return pl.pallas_call(
        _k2_body,
        grid=(GRID,),
        in_specs=[
            _view_spec(), _view_spec(), _view_spec(), _view_spec(),
            pl.BlockSpec((128, 1024), lambda i: (0, 0)),
            pl.BlockSpec((128, 1024), lambda i: (0, 0)),
            pl.BlockSpec((128, 1024), lambda i: (0, 0)),
            pl.BlockSpec((1024, 512), lambda i: (0, 0)),
        ],
        out_specs=[_vout_spec()] * 4,
        out_shape=[jax.ShapeDtypeStruct((NV, 128), f32)] * 4,
    )


def _make_k3():
    f32 = jnp.float32
    return pl.pallas_call(
        _k3_body,
        grid=(GRID,),
        in_specs=[_view_spec()] * 5,
        out_specs=[_vout_spec()] * 4,
        out_shape=[jax.ShapeDtypeStruct((NV, 128), f32)] * 4,
    )


@jax.jit
def kernel(patch_feats, patch_edge_index, patch_edge_weight, patch_graph_ids,
           mesh_edge_index, W1p, W2p, Wlp, Wcp, W1m, W2m, Wlm, blm, Wcm):
    f32 = jnp.float32
    i32 = jnp.int32
    src = patch_edge_index[0].astype(i32)
    dst = patch_edge_index[1].astype(i32)
    pad_e = E_PAD - E
    dump = jnp.full((pad_e,), N_NODES, i32)
    src_f = jnp.concatenate([src, dump])
    dst_f = jnp.concatenate([dst, dump])
    src2d = src_f.reshape(E_ROWS, 128)
    dst2d = dst_f.reshape(E_ROWS, 128)
    ew_p = jnp.concatenate(
        [patch_edge_weight.astype(f32), jnp.zeros((pad_e,), f32)])
    gid2d = jnp.concatenate(
        [patch_graph_ids.astype(i32),
         jnp.full((N_PAD - N_NODES,), NPATCH, i32)]).reshape(N_ROWS, 128)
    ms2d = mesh_edge_index[0].astype(i32).reshape(MCH, MEB)
    md2d = mesh_edge_index[1].astype(i32).reshape(MCH, MEB)
    feats_pad = jnp.concatenate(
        [patch_feats, jnp.zeros((N_PAD - N_NODES, 18), f32)])

    deg_o, deg_i, cnt = _make_sc_degrees()(src2d, dst2d, gid2d)
    degi_v = deg_i.reshape(2, NV, 128)

    xn0, xn1 = _make_k1()(feats_pad, deg_o)
    s1p0, s1p1 = _make_sc_agg(2)(xn0, xn1, src_f, dst_f, ew_p)
    eye8 = jnp.eye(8, dtype=f32)
    k1a = jnp.kron(eye8, W1p[:16])
    k1b = jnp.kron(eye8, jnp.concatenate(
        [W1p[16:18], jnp.zeros((14, 128), f32)]))
    rsel = jnp.kron(eye8, jnp.zeros((16, 128), f32).at[0].set(1.0))
    kw2 = jnp.concatenate(
        [jnp.kron(eye8, W2p[:, 16 * ch:16 * ch + 16]) for ch in range(4)],
        axis=1)
    tv = _make_k2()(
        s1p0.reshape(2, NV, 128), s1p1.reshape(2, NV, 128),
        degi_v, deg_o.reshape(2, NV, 128), k1a, k1b, rsel, kw2)
    s2p = _make_sc_agg(4)(*[t.reshape(N_PAD, 16) for t in tv],
                          src_f, dst_f, ew_p)
    h2v = _make_k3()(*[p.reshape(2, NV, 128) for p in s2p], degi_v)

    rsum = _make_sc_readout()(
        *[h.reshape(N_ROWS, 128, 16) for h in h2v], gid2d)
    readouts, logits = pl.pallas_call(
        _tail_body,
        out_shape=[jax.ShapeDtypeStruct((NPATCH, 32), f32),
                   jax.ShapeDtypeStruct((1, 15), f32)],
        compiler_params=pltpu.CompilerParams(
            vmem_limit_bytes=100 * 1024 * 1024),
    )(rsum, cnt, ms2d, md2d, Wlp, Wcp, W1m, W2m, Wlm, blm.reshape(1, 16),
      Wcm)

    return (logits, readouts)


# speedup vs baseline: 1.4198x; 1.0926x over previous
import functools

import jax
import jax.numpy as jnp
from jax import lax
from jax.experimental import pallas as pl
from jax.experimental.pallas import tpu as pltpu
from jax.experimental.pallas import tpu_sc as plsc

N_NODES = 100000
N_PAD = 102400
N_ROWS = N_PAD // 128
E = 1600000
E_PAD = 1605632
E_ROWS = E_PAD // 128
RPT = E_ROWS // 32
BR = 4
NB = RPT // BR
NPATCH = 1024
NPP = 1152
EM = 16384
EM_ROWS = EM // 128

NC, NS = 2, 16
_PROBE_NOSCALE = False
_SC_PARAMS = pltpu.CompilerParams(needs_layout_passes=False,
                                  use_tc_tiling_on_sc=False)


def _sc_mesh():
    return plsc.VectorSubcoreMesh(core_axis_name="c", subcore_axis_name="s",
                                  num_cores=NC, num_subcores=NS)


def _fill(ref, n_rows, width, value):
    @plsc.parallel_loop(0, n_rows, 1, unroll=8)
    def _(i):
        for k in range(width // 16):
            ref[i, pl.ds(16 * k, 16)] = jnp.full((16,), value, jnp.float32)


def _zero_acc(acc, s, rows_per_sub, zero_v, zrows, sem):
    base = s * rows_per_sub
    n_full = rows_per_sub // zrows

    def body(r, carry):
        pltpu.sync_copy(zero_v, acc.at[pl.ds(base + r * zrows, zrows)])
        return carry

    lax.fori_loop(0, n_full, body, 0)
    rem = rows_per_sub - n_full * zrows
    if rem:
        pltpu.sync_copy(zero_v.at[pl.ds(0, rem)],
                        acc.at[pl.ds(base + n_full * zrows, rem)])


def _dump_acc(acc, out, c, s, rows_per_sub):
    base = s * rows_per_sub
    pltpu.sync_copy(acc.at[pl.ds(base, rows_per_sub)],
                    out.at[c].at[pl.ds(base, rows_per_sub)])


def _sc_degrees_body(src2d, dst2d, gid2d,
                     deg_o, deg_i, cnt,
                     idx_v, ones_v, zero_v, acc, sem_a):
    c = lax.axis_index("c")
    s = lax.axis_index("s")
    w = c * NS + s
    _fill(ones_v, 128, 16, 1.0)
    _fill(zero_v, 128, 16, 0.0)

    def hist_pass(idx2d, out, tile_rows, nrows_acc):
        rps = nrows_acc // NS
        _zero_acc(acc, s, rps, zero_v, 128, sem_a)
        plsc.subcore_barrier()
        n_full = tile_rows // 8

        def batch(b, carry):
            pltpu.sync_copy(idx2d.at[pl.ds(w * tile_rows + b * 8, 8)], idx_v)
            descs = [pltpu.async_copy(ones_v, acc.at[idx_v.at[j]], sem_a,
                                      add=True) for j in range(8)]
            for d in descs:
                d.wait()
            return carry

        lax.fori_loop(0, n_full, batch, 0)
        rem = tile_rows - n_full * 8
        if rem:
            pltpu.sync_copy(
                idx2d.at[pl.ds(w * tile_rows + n_full * 8, rem)],
                idx_v.at[pl.ds(0, rem)])
            descs = [pltpu.async_copy(ones_v, acc.at[idx_v.at[j]], sem_a,
                                      add=True) for j in range(rem)]
            for d in descs:
                d.wait()
        plsc.subcore_barrier()
        _dump_acc(acc, out, c, s, rps)
        plsc.subcore_barrier()

    hist_pass(src2d, deg_o, RPT, N_PAD)
    hist_pass(dst2d, deg_i, RPT, N_PAD)
    hist_pass(gid2d, cnt, N_ROWS // 32, NPP)


def _make_sc_degrees():
    f32 = jnp.float32
    return pl.kernel(
        _sc_degrees_body,
        out_type=(
            jax.ShapeDtypeStruct((NC, N_PAD, 16), f32),
            jax.ShapeDtypeStruct((NC, N_PAD, 16), f32),
            jax.ShapeDtypeStruct((NC, NPP, 16), f32),
        ),
        mesh=_sc_mesh(),
        scratch_types=[
            pltpu.VMEM((8, 128), jnp.int32),
            pltpu.VMEM((128, 16), f32),
            pltpu.VMEM((128, 16), f32),
            pltpu.VMEM_SHARED((N_PAD, 16), f32),
            pltpu.SemaphoreType.DMA,
        ],
        compiler_params=_SC_PARAMS,
    )


def _make_sc_agg(nchunks):
    f32 = jnp.float32

    def body(*refs):
        tables = refs[:nchunks]
        srcf, dstf, ew = refs[nchunks:nchunks + 3]
        outs = refs[nchunks + 3:2 * nchunks + 3]
        (sidx_a, sidx_b, didx_a, didx_b, ew_a, ew_b, rows_a, rows_b,
         zero_v, acc, sem_g, sem_a) = refs[2 * nchunks + 3:]
        c = lax.axis_index("c")
        s = lax.axis_index("s")
        w = c * NS + s
        _fill(zero_v, 128, 16, 0.0)
        e0 = w * (RPT * 128)
        bsz = BR * 128

        def load(b, sidx, didx, ewv):
            pltpu.sync_copy(srcf.at[pl.ds(e0 + b * bsz, bsz)], sidx)
            pltpu.sync_copy(dstf.at[pl.ds(e0 + b * bsz, bsz)], didx)
            pltpu.sync_copy(ew.at[pl.ds(e0 + b * bsz, bsz)], ewv)

        def scale(rows, ewv):
            if _PROBE_NOSCALE:
                return

            @plsc.parallel_loop(0, BR * 128, 1, unroll=4)
            def _(e):
                sc = plsc.load_gather(ewv, [jnp.full((16,), e, jnp.int32)])
                rows[e, :] = rows[e, :] * sc

        for tbl, out in zip(tables, outs):
            def fire_g(sidx, rows):
                pltpu.async_copy(tbl.at[sidx], rows, sem_g)

            def drain_g(sidx, rows):
                pltpu.make_async_copy(tbl.at[sidx], rows, sem_g).wait()

            def fire_a(didx, rows):
                pltpu.async_copy(rows, acc.at[didx], sem_a, add=True)

            def drain_a(didx, rows):
                pltpu.make_async_copy(rows, acc.at[didx], sem_a).wait()

            _zero_acc(acc, s, N_PAD // NS, zero_v, 128, sem_a)
            plsc.subcore_barrier()

            load(0, sidx_a, didx_a, ew_a)
            fire_g(sidx_a, rows_a)

            def pair(bb, load_next):
                b0 = 2 * bb
                drain_g(sidx_a, rows_a)

                @pl.when(bb > 0)
                def _():
                    drain_a(didx_b, rows_b)

                load(b0 + 1, sidx_b, didx_b, ew_b)
                fire_g(sidx_b, rows_b)
                scale(rows_a, ew_a)
                fire_a(didx_a, rows_a)
                drain_g(sidx_b, rows_b)
                drain_a(didx_a, rows_a)
                if load_next:
                    load(b0 + 2, sidx_a, didx_a, ew_a)
                    fire_g(sidx_a, rows_a)
                scale(rows_b, ew_b)
                fire_a(didx_b, rows_b)

            def pair_body(bb, carry):
                pair(bb, True)
                return carry

            lax.fori_loop(0, NB // 2 - 1, pair_body, 0)
            pair(NB // 2 - 1, False)
            drain_a(didx_b, rows_b)
            plsc.subcore_barrier()
            _dump_acc(acc, out, c, s, N_PAD // NS)
            plsc.subcore_barrier()

    f32 = jnp.float32
    return pl.kernel(
        body,
        out_type=tuple(jax.ShapeDtypeStruct((NC, N_PAD, 16), f32)
                       for _ in range(nchunks)),
        mesh=_sc_mesh(),
        scratch_types=[
            pltpu.VMEM((BR * 128,), jnp.int32),
            pltpu.VMEM((BR * 128,), jnp.int32),
            pltpu.VMEM((BR * 128,), jnp.int32),
            pltpu.VMEM((BR * 128,), jnp.int32),
            pltpu.VMEM((BR * 128,), f32),
            pltpu.VMEM((BR * 128,), f32),
            pltpu.VMEM((BR * 128, 16), f32),
            pltpu.VMEM((BR * 128, 16), f32),
            pltpu.VMEM((128, 16), f32),
            pltpu.VMEM_SHARED((N_PAD, 16), f32),
            pltpu.SemaphoreType.DMA,
            pltpu.SemaphoreType.DMA,
        ],
        compiler_params=_SC_PARAMS,
    )


def _sc_readout_body(h0, h1, h2, h3, gid2d, out,
                     idx_v, vals_v, zero_v, a0, a1, a2, a3, sem_a):
    c = lax.axis_index("c")
    s = lax.axis_index("s")
    w = c * NS + s
    tile_rows = N_ROWS // 32
    _fill(zero_v, 72, 16, 0.0)
    rps = NPP // NS
    base = s * rps
    accs = (a0, a1, a2, a3)
    for acc in accs:
        pltpu.sync_copy(zero_v, acc.at[pl.ds(base, rps)])
    plsc.subcore_barrier()
    pltpu.sync_copy(gid2d.at[pl.ds(w * tile_rows, tile_rows)], idx_v)
    for hsrc, acc in zip((h0, h1, h2, h3), accs):
        for b in range(5):
            pltpu.sync_copy(hsrc.at[pl.ds(w * tile_rows + b * 5, 5)], vals_v)
            descs = [pltpu.async_copy(vals_v.at[j],
                                      acc.at[idx_v.at[b * 5 + j]],
                                      sem_a, add=True) for j in range(5)]
            for d in descs:
                d.wait()
    plsc.subcore_barrier()
    for ch, acc in enumerate(accs):
        pltpu.sync_copy(acc.at[pl.ds(base, rps)],
                        out.at[c].at[ch].at[pl.ds(base, rps)])


def _make_sc_readout():
    f32 = jnp.float32
    return pl.kernel(
        _sc_readout_body,
        out_type=jax.ShapeDtypeStruct((NC, 4, NPP, 16), f32),
        mesh=_sc_mesh(),
        scratch_types=[
            pltpu.VMEM((25, 128), jnp.int32),
            pltpu.VMEM((5, 128, 16), f32),
            pltpu.VMEM((72, 16), f32),
            pltpu.VMEM_SHARED((NPP, 16), f32),
            pltpu.VMEM_SHARED((NPP, 16), f32),
            pltpu.VMEM_SHARED((NPP, 16), f32),
            pltpu.VMEM_SHARED((NPP, 16), f32),
            pltpu.SemaphoreType.DMA,
        ],
        compiler_params=_SC_PARAMS,
    )


BLK = 2048
GRID = N_PAD // BLK


def _leaky(x):
    return jnp.where(x >= 0, x, 0.01 * x)


def _norm(deg_ref):
    d = deg_ref[0, :, 0:1] + deg_ref[1, :, 0:1]
    return lax.rsqrt(jnp.maximum(d, 1.0))


def _k1_body(fv, degov, t0a, t1a, t0b, t1b, rsel16, xn0, xn1):
    f32 = jnp.float32
    no16 = jnp.dot(_norm_view(degov), rsel16[...],
                   preferred_element_type=f32)
    fr = fv[...].reshape(256, 2, 128)
    e = fr[:, 0, :]
    o = fr[:, 1, :]
    xn0[...] = (jnp.dot(e, t0a[...], preferred_element_type=f32)
                + jnp.dot(o, t1a[...], preferred_element_type=f32)) * no16
    xn1[...] = (jnp.dot(e, t0b[...], preferred_element_type=f32)
                + jnp.dot(o, t1b[...], preferred_element_type=f32)) * no16


def _norm_view(deg_ref):
    return lax.rsqrt(jnp.maximum(deg_ref[0] + deg_ref[1], 1.0))


def _k2_body(s1p0v, s1p1v, degiv, degov, k1a, k1b, rsel, kw2, t0, t1, t2, t3):
    f32 = jnp.float32
    ni = _norm_view(degiv)
    h_v = jnp.dot((s1p0v[0] + s1p0v[1]) * ni, k1a[...],
                  preferred_element_type=f32)
    h_v = h_v + jnp.dot((s1p1v[0] + s1p1v[1]) * ni, k1b[...],
                        preferred_element_type=f32)
    no_v = jnp.dot(_norm_view(degov), rsel[...], preferred_element_type=f32)
    t_v = jnp.dot(_leaky(h_v) * no_v, kw2[...], preferred_element_type=f32)
    t0[...] = t_v[:, 0:128]
    t1[...] = t_v[:, 128:256]
    t2[...] = t_v[:, 256:384]
    t3[...] = t_v[:, 384:512]


def _k3_body(p0, p1, p2, p3, degiv, h0, h1, h2, h3):
    ni = _norm_view(degiv)
    for p, o in ((p0, h0), (p1, h1), (p2, h2), (p3, h3)):
        o[...] = _leaky((p[0] + p[1]) * ni)


MEB = 512
MCH = EM // MEB


def _mesh_onehot(idx_row):
    cols = lax.broadcasted_iota(jnp.int32, (MEB, NPATCH), 1)
    return jnp.where(idx_row.reshape(MEB, 1) == cols, 1.0, 0.0)


def _tail_body(rsum, cnt, ms2d, md2d, wl, wc, w1m, w2m, wlm, blm, wcm,
               readouts, logits):
    f32 = jnp.float32
    sums = jnp.concatenate(
        [rsum[0, ch, :NPATCH] + rsum[1, ch, :NPATCH] for ch in range(4)],
        axis=1)
    c = cnt[0, :NPATCH, 0:1] + cnt[1, :NPATCH, 0:1]
    r = jnp.dot(jnp.dot(sums / jnp.maximum(c, 1.0), wl[...],
                        preferred_element_type=f32), wc[...],
                preferred_element_type=f32)
    readouts[...] = r

    mdo = jnp.zeros((NPATCH,), f32)
    mdi = jnp.zeros((NPATCH,), f32)
    for k in range(MCH):
        mdo = mdo + jnp.sum(_mesh_onehot(ms2d[k]), axis=0)
        mdi = mdi + jnp.sum(_mesh_onehot(md2d[k]), axis=0)
    mno = lax.rsqrt(jnp.maximum(mdo, 1.0)).reshape(NPATCH, 1)
    mni = lax.rsqrt(jnp.maximum(mdi, 1.0)).reshape(NPATCH, 1)

    def mesh_conv(x):
        agg = jnp.zeros((NPATCH, x.shape[1]), f32)
        for k in range(MCH):
            msgs = jnp.dot(_mesh_onehot(ms2d[k]), x, preferred_element_type=f32)
            agg = agg + jnp.dot(_mesh_onehot(md2d[k]).T, msgs,
                                preferred_element_type=f32)
        return agg

    xm = r * mno
    g1 = _leaky(mesh_conv(xm) * mni @ w1m[...])
    xm2 = (g1 * mno) @ w2m[...]
    g2 = _leaky(mesh_conv(xm2) * mni)
    z = jnp.dot(g2, wlm[...], preferred_element_type=f32) + blm[...]
    pooled = jnp.mean(z, axis=0, keepdims=True)
    logits[...] = jnp.dot(pooled, wcm[...], preferred_element_type=f32)


def _deg_spec():
    return pl.BlockSpec((2, BLK, 16), lambda i: (0, i, 0))


def _chunk_spec():
    return pl.BlockSpec((BLK, 16), lambda i: (i, 0))


def _pair_spec():
    return pl.BlockSpec((2, BLK, 16), lambda i: (0, i, 0))


def _make_k1():
    f32 = jnp.float32
    wspec = pl.BlockSpec((128, 128), lambda i: (0, 0))
    return pl.pallas_call(
        _k1_body,
        grid=(GRID,),
        in_specs=[pl.BlockSpec((BLK // 4, 128), lambda i: (i, 0)),
                  _view_spec(), wspec, wspec, wspec, wspec, wspec],
        out_specs=[_vout_spec(), _vout_spec()],
        out_shape=[jax.ShapeDtypeStruct((NV, 128), f32)] * 2,
    )


VBLK = BLK // 8
NV = N_PAD // 8


def _view_spec():
    return pl.BlockSpec((2, VBLK, 128), lambda i: (0, i, 0))


def _vout_spec():
    return pl.BlockSpec((VBLK, 128), lambda i: (i, 0))


def _make_k2():
    f32 = jnp.float32
    return pl.pallas_call(
        _k2_body,
        grid=(GRID,),
        in_specs=[
            _view_spec(), _view_spec(), _view_spec(), _view_spec(),
            pl.BlockSpec((128, 1024), lambda i: (0, 0)),
            pl.BlockSpec((128, 1024), lambda i: (0, 0)),
            pl.BlockSpec((128, 1024), lambda i: (0, 0)),
            pl.BlockSpec((1024, 512), lambda i: (0, 0)),
        ],
        out_specs=[_vout_spec()] * 4,
        out_shape=[jax.ShapeDtypeStruct((NV, 128), f32)] * 4,
    )


def _make_k3():
    f32 = jnp.float32
    return pl.pallas_call(
        _k3_body,
        grid=(GRID,),
        in_specs=[_view_spec()] * 5,
        out_specs=[_vout_spec()] * 4,
        out_shape=[jax.ShapeDtypeStruct((NV, 128), f32)] * 4,
    )


@jax.jit
def kernel(patch_feats, patch_edge_index, patch_edge_weight, patch_graph_ids,
           mesh_edge_index, W1p, W2p, Wlp, Wcp, W1m, W2m, Wlm, blm, Wcm):
    f32 = jnp.float32
    i32 = jnp.int32
    src = patch_edge_index[0].astype(i32)
    dst = patch_edge_index[1].astype(i32)
    pad_e = E_PAD - E
    dump = jnp.full((pad_e,), N_NODES, i32)
    src_f = jnp.concatenate([src, dump])
    dst_f = jnp.concatenate([dst, dump])
    src2d = src_f.reshape(E_ROWS, 128)
    dst2d = dst_f.reshape(E_ROWS, 128)
    ew_p = jnp.concatenate(
        [patch_edge_weight.astype(f32), jnp.zeros((pad_e,), f32)])
    gid2d = jnp.concatenate(
        [patch_graph_ids.astype(i32),
         jnp.full((N_PAD - N_NODES,), NPATCH, i32)]).reshape(N_ROWS, 128)
    ms2d = mesh_edge_index[0].astype(i32).reshape(MCH, MEB)
    md2d = mesh_edge_index[1].astype(i32).reshape(MCH, MEB)
    feats_v = jnp.pad(patch_feats,
                      ((0, N_PAD - N_NODES), (0, 14))).reshape(N_PAD // 4, 128)

    deg_o, deg_i, cnt = _make_sc_degrees()(src2d, dst2d, gid2d)
    degi_v = deg_i.reshape(2, NV, 128)
    dego_v = deg_o.reshape(2, NV, 128)

    cc = jnp.arange(4)[:, None]
    jj = jnp.arange(16)[None, :]
    j2 = jnp.arange(2)[None, :]
    z128 = jnp.zeros((128, 128), f32)
    t0a = z128.at[(32 * cc + jj).ravel(), (16 * cc + jj).ravel()].set(1.0)
    t1a = z128.at[(32 * cc + jj).ravel(),
                  (16 * (cc + 4) + jj).ravel()].set(1.0)
    t0b = z128.at[(32 * cc + 16 + j2).ravel(), (16 * cc + j2).ravel()].set(1.0)
    t1b = z128.at[(32 * cc + 16 + j2).ravel(),
                  (16 * (cc + 4) + j2).ravel()].set(1.0)
    rsel16 = jnp.kron(jnp.eye(8, dtype=f32),
                      jnp.zeros((16, 16), f32).at[0].set(1.0))

    xn0, xn1 = _make_k1()(feats_v, dego_v, t0a, t1a, t0b, t1b, rsel16)
    s1p0, s1p1 = _make_sc_agg(2)(xn0.reshape(N_PAD, 16),
                                 xn1.reshape(N_PAD, 16), src_f, dst_f, ew_p)
    eye8 = jnp.eye(8, dtype=f32)
    k1a = jnp.kron(eye8, W1p[:16])
    k1b = jnp.kron(eye8, jnp.concatenate(
        [W1p[16:18], jnp.zeros((14, 128), f32)]))
    rsel = jnp.kron(eye8, jnp.zeros((16, 128), f32).at[0].set(1.0))
    kw2 = jnp.concatenate(
        [jnp.kron(eye8, W2p[:, 16 * ch:16 * ch + 16]) for ch in range(4)],
        axis=1)
    tv = _make_k2()(
        s1p0.reshape(2, NV, 128), s1p1.reshape(2, NV, 128),
        degi_v, dego_v, k1a, k1b, rsel, kw2)
    s2p = _make_sc_agg(4)(*[t.reshape(N_PAD, 16) for t in tv],
                          src_f, dst_f, ew_p)
    h2v = _make_k3()(*[p.reshape(2, NV, 128) for p in s2p], degi_v)

    rsum = _make_sc_readout()(
        *[h.reshape(N_ROWS, 128, 16) for h in h2v], gid2d)
    readouts, logits = pl.pallas_call(
        _tail_body,
        out_shape=[jax.ShapeDtypeStruct((NPATCH, 32), f32),
                   jax.ShapeDtypeStruct((1, 15), f32)],
        compiler_params=pltpu.CompilerParams(
            vmem_limit_bytes=100 * 1024 * 1024),
    )(rsum, cnt, ms2d, md2d, Wlp, Wcp, W1m, W2m, Wlm, blm.reshape(1, 16),
      Wcm)

    return (logits, readouts)
